# baseline (speedup 1.0000x reference)
import functools

import jax
import jax.numpy as jnp
from jax import lax
from jax.experimental import pallas as pl
from jax.experimental.pallas import tpu as pltpu
from jax.experimental.pallas import tpu_sc as plsc

DIM = 16
NUM_CORES = 2
NUM_SUBCORES = 16
NUM_WORKERS = NUM_CORES * NUM_SUBCORES
REG_COEF = 0.001


def _make_sc_gather(batch):
    b_per_w = batch // NUM_WORKERS
    mesh = plsc.VectorSubcoreMesh(core_axis_name="c", subcore_axis_name="s")

    @functools.partial(
        pl.kernel,
        mesh=mesh,
        out_type=(
            jax.ShapeDtypeStruct((batch, DIM), jnp.float32),
            jax.ShapeDtypeStruct((batch, DIM), jnp.float32),
        ),
        scratch_types=[
            pltpu.VMEM((b_per_w,), jnp.int32),
            pltpu.VMEM((b_per_w, DIM), jnp.float32),
            pltpu.VMEM((b_per_w,), jnp.int32),
            pltpu.VMEM((b_per_w, DIM), jnp.float32),
            pltpu.SemaphoreType.DMA,
            pltpu.SemaphoreType.DMA,
        ],
        compiler_params=pltpu.CompilerParams(use_tc_tiling_on_sc=False),
    )
    def gather_kernel(users_hbm, items_hbm, utab_hbm, itab_hbm,
                      u_out, v_out, uidx_v, urows_v, iidx_v, irows_v,
                      sem_u, sem_v):
        wid = lax.axis_index("s") * NUM_CORES + lax.axis_index("c")
        base = wid * b_per_w
        pltpu.sync_copy(users_hbm.at[pl.ds(base, b_per_w)], uidx_v)
        pltpu.sync_copy(items_hbm.at[pl.ds(base, b_per_w)], iidx_v)
        cu = pltpu.async_copy(utab_hbm.at[uidx_v], urows_v, sem_u)
        ci = pltpu.async_copy(itab_hbm.at[iidx_v], irows_v, sem_v)
        cu.wait()
        ci.wait()
        pltpu.sync_copy(urows_v, u_out.at[pl.ds(base, b_per_w)])
        pltpu.sync_copy(irows_v, v_out.at[pl.ds(base, b_per_w)])

    return gather_kernel


def _dot_reg_body(u_ref, v_ref, inf_ref, reg_ref):
    u = u_ref[...]
    v = v_ref[...]
    inf_ref[...] = jnp.sum(u * v, axis=1, keepdims=True)
    reg_ref[0, 0] = REG_COEF * (jnp.sum(u * u) + jnp.sum(v * v))


def kernel(users, items, user_table, item_table):
    batch = users.shape[0]
    users = users.astype(jnp.int32)
    items = items.astype(jnp.int32)
    u_rows, v_rows = _make_sc_gather(batch)(users, items, user_table, item_table)

    inferences, regs = pl.pallas_call(
        _dot_reg_body,
        out_shape=(
            jax.ShapeDtypeStruct((batch, 1), jnp.float32),
            jax.ShapeDtypeStruct((1, 1), jnp.float32),
        ),
        out_specs=(
            pl.BlockSpec(memory_space=pltpu.VMEM),
            pl.BlockSpec(memory_space=pltpu.SMEM),
        ),
    )(u_rows, v_rows)
    return inferences, regs[0, 0]


# speedup vs baseline: 1.0245x; 1.0245x over previous
import functools

import jax
import jax.numpy as jnp
from jax import lax
from jax.experimental import pallas as pl
from jax.experimental.pallas import tpu as pltpu
from jax.experimental.pallas import tpu_sc as plsc

DIM = 16
LANES = 16
WIDE = 128
ROWS_PER_WIDE = WIDE // DIM
NUM_CORES = 2
NUM_SUBCORES = 16
NUM_WORKERS = NUM_CORES * NUM_SUBCORES
CHUNK = 128
REG_COEF = 0.001


def _make_sc_fused(batch):
    b_per_w = batch // NUM_WORKERS
    n_chunks = b_per_w // CHUNK
    mesh = plsc.VectorSubcoreMesh(core_axis_name="c", subcore_axis_name="s")

    @functools.partial(
        pl.kernel,
        mesh=mesh,
        out_type=(
            jax.ShapeDtypeStruct((batch,), jnp.float32),
            jax.ShapeDtypeStruct((NUM_WORKERS, LANES), jnp.float32),
        ),
        scratch_types=[
            pltpu.VMEM((b_per_w,), jnp.int32),
            pltpu.VMEM((b_per_w,), jnp.int32),
            pltpu.VMEM((b_per_w,), jnp.int32),
            pltpu.VMEM((b_per_w,), jnp.int32),
            pltpu.VMEM((CHUNK, WIDE), jnp.float32),
            pltpu.VMEM((CHUNK, WIDE), jnp.float32),
            pltpu.VMEM((CHUNK, WIDE), jnp.float32),
            pltpu.VMEM((CHUNK, WIDE), jnp.float32),
            pltpu.VMEM((b_per_w,), jnp.float32),
            pltpu.VMEM((LANES,), jnp.float32),
            pltpu.SemaphoreType.DMA,
            pltpu.SemaphoreType.DMA,
            pltpu.SemaphoreType.DMA,
            pltpu.SemaphoreType.DMA,
        ],
        compiler_params=pltpu.CompilerParams(needs_layout_passes=False),
    )
    def sc_kernel(users_hbm, items_hbm, utab_hbm, itab_hbm,
                  inf_out, reg_out,
                  uidx_v, iidx_v, uwid_v, iwid_v,
                  ubuf0, ubuf1, vbuf0, vbuf1,
                  inf_v, racc_v, semu0, semu1, semv0, semv1):
        ubufs, vbufs = (ubuf0, ubuf1), (vbuf0, vbuf1)
        semus, semvs = (semu0, semu1), (semv0, semv1)
        wid = lax.axis_index("s") * NUM_CORES + lax.axis_index("c")
        base = wid * b_per_w

        pltpu.sync_copy(users_hbm.at[pl.ds(base, b_per_w)], uidx_v)
        pltpu.sync_copy(items_hbm.at[pl.ds(base, b_per_w)], iidx_v)

        @pl.loop(0, b_per_w, step=LANES)
        def _(t):
            sl = pl.ds(t, LANES)
            uwid_v[sl] = lax.shift_right_logical(uidx_v[sl], 3)
            iwid_v[sl] = lax.shift_right_logical(iidx_v[sl], 3)

        racc_v[...] = jnp.zeros((LANES,), jnp.float32)
        iota = lax.iota(jnp.int32, LANES)

        def fire(k):
            sl = pl.ds(k * CHUNK, CHUNK)
            cu = pltpu.async_copy(utab_hbm.at[uwid_v.at[sl]],
                                  ubufs[k % 2], semus[k % 2])
            cv = pltpu.async_copy(itab_hbm.at[iwid_v.at[sl]],
                                  vbufs[k % 2], semvs[k % 2])
            return cu, cv

        inflight = {0: fire(0)}
        for k in range(n_chunks):
            if k + 1 < n_chunks:
                inflight[k + 1] = fire(k + 1)
            cu, cv = inflight.pop(k)
            cu.wait()
            cv.wait()
            ubuf, vbuf = ubufs[k % 2], vbufs[k % 2]

            @pl.loop(0, CHUNK, step=LANES)
            def _(g):
                row16 = g + iota
                pu = (uidx_v[pl.ds(k * CHUNK + g, LANES)] & 7) << 4
                qv = (iidx_v[pl.ds(k * CHUNK + g, LANES)] & 7) << 4
                acc = jnp.zeros((LANES,), jnp.float32)
                rloc = jnp.zeros((LANES,), jnp.float32)
                for l in range(DIM):
                    cu16 = plsc.load_gather(ubuf, [row16, pu + l])
                    cv16 = plsc.load_gather(vbuf, [row16, qv + l])
                    acc = acc + cu16 * cv16
                    rloc = rloc + (cu16 * cu16 + cv16 * cv16)
                inf_v[pl.ds(k * CHUNK + g, LANES)] = acc
                racc_v[...] = racc_v[...] + rloc

        pltpu.sync_copy(inf_v, inf_out.at[pl.ds(base, b_per_w)])
        pltpu.sync_copy(racc_v, reg_out.at[wid])

    return sc_kernel


def _reg_body(p_ref, out_ref):
    out_ref[0, 0] = REG_COEF * jnp.sum(p_ref[...])


def kernel(users, items, user_table, item_table):
    batch = users.shape[0]
    users = users.astype(jnp.int32)
    items = items.astype(jnp.int32)
    inf, reg_partials = _make_sc_fused(batch)(
        users, items,
        user_table.reshape(-1, WIDE), item_table.reshape(-1, WIDE))

    regs = pl.pallas_call(
        _reg_body,
        out_shape=jax.ShapeDtypeStruct((1, 1), jnp.float32),
        out_specs=pl.BlockSpec(memory_space=pltpu.SMEM),
    )(reg_partials)
    return inf.reshape(batch, 1), regs[0, 0]


# speedup vs baseline: 1.5324x; 1.4958x over previous
import functools

import jax
import jax.numpy as jnp
from jax import lax
from jax.experimental import pallas as pl
from jax.experimental.pallas import tpu as pltpu
from jax.experimental.pallas import tpu_sc as plsc

DIM = 16
LANES = 16
NUM_CORES = 2
NUM_SUBCORES = 16
NUM_WORKERS = NUM_CORES * NUM_SUBCORES
HALF = 128
REG_COEF = 0.001


def _make_sc_fused(batch):
    b_per_w = batch // NUM_WORKERS
    n_halves = b_per_w // HALF
    mesh = plsc.VectorSubcoreMesh(core_axis_name="c", subcore_axis_name="s")

    @functools.partial(
        pl.kernel,
        mesh=mesh,
        out_type=(
            jax.ShapeDtypeStruct((batch,), jnp.float32),
            jax.ShapeDtypeStruct((NUM_WORKERS, LANES), jnp.float32),
        ),
        scratch_types=[
            pltpu.VMEM((b_per_w,), jnp.int32),
            pltpu.VMEM((b_per_w,), jnp.int32),
            pltpu.VMEM((HALF, DIM), jnp.float32),
            pltpu.VMEM((HALF, DIM), jnp.float32),
            pltpu.VMEM((HALF, DIM), jnp.float32),
            pltpu.VMEM((HALF, DIM), jnp.float32),
            pltpu.VMEM((b_per_w,), jnp.float32),
            pltpu.VMEM((LANES,), jnp.float32),
            pltpu.SemaphoreType.DMA,
            pltpu.SemaphoreType.DMA,
            pltpu.SemaphoreType.DMA,
            pltpu.SemaphoreType.DMA,
        ],
        compiler_params=pltpu.CompilerParams(needs_layout_passes=False),
    )
    def sc_kernel(users_hbm, items_hbm, utab_hbm, itab_hbm,
                  inf_out, reg_out,
                  uidx_v, iidx_v, ubuf0, ubuf1, vbuf0, vbuf1,
                  inf_v, racc_v, semu0, semu1, semv0, semv1):
        ubufs, vbufs = (ubuf0, ubuf1), (vbuf0, vbuf1)
        semus, semvs = (semu0, semu1), (semv0, semv1)
        wid = lax.axis_index("s") * NUM_CORES + lax.axis_index("c")
        base = wid * b_per_w

        pltpu.sync_copy(users_hbm.at[pl.ds(base, b_per_w)], uidx_v)
        pltpu.sync_copy(items_hbm.at[pl.ds(base, b_per_w)], iidx_v)

        racc_v[...] = jnp.zeros((LANES,), jnp.float32)
        iota = lax.iota(jnp.int32, LANES)

        def fire(h):
            ubuf, vbuf = ubufs[h % 2], vbufs[h % 2]
            semu, semv = semus[h % 2], semvs[h % 2]

            @pl.loop(0, HALF, step=LANES)
            def _(t):
                uvec = uidx_v[pl.ds(h * HALF + t, LANES)]
                ivec = iidx_v[pl.ds(h * HALF + t, LANES)]
                for k in range(LANES):
                    pltpu.async_copy(utab_hbm.at[uvec[k]], ubuf.at[t + k],
                                     semu)
                    pltpu.async_copy(itab_hbm.at[ivec[k]], vbuf.at[t + k],
                                     semv)

        def drain(h):
            semu, semv = semus[h % 2], semvs[h % 2]

            @pl.loop(0, HALF)
            def _(j):
                pltpu.make_async_copy(utab_hbm.at[0], ubufs[h % 2].at[0],
                                      semu).wait()
                pltpu.make_async_copy(itab_hbm.at[0], vbufs[h % 2].at[0],
                                      semv).wait()

        def compute(h):
            ubuf, vbuf = ubufs[h % 2], vbufs[h % 2]

            @pl.loop(0, HALF, step=LANES)
            def _(t):
                row16 = t + iota
                acc = jnp.zeros((LANES,), jnp.float32)
                rloc = jnp.zeros((LANES,), jnp.float32)
                for l in range(DIM):
                    lane = jnp.full((LANES,), l, jnp.int32)
                    cu16 = plsc.load_gather(ubuf, [row16, lane])
                    cv16 = plsc.load_gather(vbuf, [row16, lane])
                    acc = acc + cu16 * cv16
                    rloc = rloc + (cu16 * cu16 + cv16 * cv16)
                inf_v[pl.ds(h * HALF + t, LANES)] = acc
                racc_v[...] = racc_v[...] + rloc

        fire(0)
        for h in range(n_halves):
            if h + 1 < n_halves:
                fire(h + 1)
            drain(h)
            compute(h)

        pltpu.sync_copy(inf_v, inf_out.at[pl.ds(base, b_per_w)])
        pltpu.sync_copy(racc_v, reg_out.at[wid])

    return sc_kernel


def _reg_body(p_ref, out_ref):
    out_ref[0, 0] = REG_COEF * jnp.sum(p_ref[...])


def kernel(users, items, user_table, item_table):
    batch = users.shape[0]
    users = users.astype(jnp.int32)
    items = items.astype(jnp.int32)
    inf, reg_partials = _make_sc_fused(batch)(
        users, items, user_table, item_table)

    regs = pl.pallas_call(
        _reg_body,
        out_shape=jax.ShapeDtypeStruct((1, 1), jnp.float32),
        out_specs=pl.BlockSpec(memory_space=pltpu.SMEM),
    )(reg_partials)
    return inf.reshape(batch, 1), regs[0, 0]
